# Initial kernel scaffold; baseline (speedup 1.0000x reference)
import jax
import jax.numpy as jnp
from jax.experimental import pallas as pl


def kernel(piece_x, square_x, params, ei_ps, ei_sp, ei_ss, piece_batch, square_batch):
    raise NotImplementedError("write your pallas kernel here")


# speedup vs baseline: 7.9747x; 7.9747x over previous
import functools

import numpy as np
import jax
import jax.numpy as jnp
from jax import lax
from jax.experimental import pallas as pl
from jax.experimental.pallas import tpu as pltpu
from jax.experimental.pallas import tpu_sc as plsc

F32 = jnp.float32
H = 4
DH = 32
D = 128
NC, NS = 2, 16
NW = NC * NS
CH = 128
ZC = 200
PREC = lax.Precision.HIGHEST

_MASK16 = np.zeros((D, 16), np.float32)
for _h in range(H):
    _MASK16[_h * DH:(_h + 1) * DH, _h] = 1.0
_T16 = np.zeros((16, D), np.float32)
for _h in range(H):
    _T16[_h, _h * DH:(_h + 1) * DH] = 1.0


def _proj(x, w, b, nout, gelu, blk=400):
    n, kdim = x.shape

    def body(x_ref, w_ref, b_ref, *out_refs):
        acc = jnp.dot(x_ref[...], w_ref[...], precision=PREC) + b_ref[...]
        if gelu:
            acc = jax.nn.gelu(acc)
        for j, o in enumerate(out_refs):
            o[...] = acc[:, j * D:(j + 1) * D]

    return pl.pallas_call(
        body,
        grid=(n // blk,),
        in_specs=[
            pl.BlockSpec((blk, kdim), lambda i: (i, 0)),
            pl.BlockSpec((kdim, nout * D), lambda i: (0, 0)),
            pl.BlockSpec((1, nout * D), lambda i: (0, 0)),
        ],
        out_specs=[pl.BlockSpec((blk, D), lambda i: (i, 0))
                   for _ in range(nout)],
        out_shape=[jax.ShapeDtypeStruct((n, D), F32) for _ in range(nout)],
    )(x, w, b)


def _combine(wstack, bstack, astack):
    s = wstack.shape[0]

    def body(w_ref, b_ref, a_ref, ow_ref, ob_ref):
        for h in range(H):
            sl = slice(h * DH, (h + 1) * DH)
            a = a_ref[0, h]
            ow_ref[0, :, sl] = jnp.dot(w_ref[0][:, sl], a, precision=PREC)
            ob_ref[0, :, sl] = jnp.dot(b_ref[0][:, sl], a, precision=PREC)

    return pl.pallas_call(
        body,
        grid=(s,),
        in_specs=[
            pl.BlockSpec((1, D, D), lambda i: (i, 0, 0)),
            pl.BlockSpec((1, 1, D), lambda i: (i, 0, 0)),
            pl.BlockSpec((1, H, DH, DH), lambda i: (i, 0, 0, 0)),
        ],
        out_specs=[
            pl.BlockSpec((1, D, D), lambda i: (i, 0, 0)),
            pl.BlockSpec((1, 1, D), lambda i: (i, 0, 0)),
        ],
        out_shape=[jax.ShapeDtypeStruct((s, D, D), F32),
                   jax.ShapeDtypeStruct((s, 1, D), F32)],
    )(wstack, bstack, astack)


def _edge_compute(rka, rq, rva, smat, tmat, blk=640):
    e = rka.shape[0]

    def body(ka_ref, q_ref, va_ref, s_ref, t_ref, c0_ref, c1_ref, c2_ref,
             c3_ref, ex_ref):
        prod = ka_ref[...] * q_ref[...]
        alpha = jnp.dot(prod, s_ref[...], precision=PREC)
        ex = jnp.exp(alpha)
        exb = jnp.dot(ex, t_ref[...], precision=PREC)
        contrib = va_ref[...] * exb
        c0_ref[...] = contrib[:, 0:32]
        c1_ref[...] = contrib[:, 32:64]
        c2_ref[...] = contrib[:, 64:96]
        c3_ref[...] = contrib[:, 96:128]
        ex_ref[...] = ex

    return pl.pallas_call(
        body,
        grid=(e // blk,),
        in_specs=[
            pl.BlockSpec((blk, D), lambda i: (i, 0)),
            pl.BlockSpec((blk, D), lambda i: (i, 0)),
            pl.BlockSpec((blk, D), lambda i: (i, 0)),
            pl.BlockSpec((D, 16), lambda i: (0, 0)),
            pl.BlockSpec((16, D), lambda i: (0, 0)),
        ],
        out_specs=[pl.BlockSpec((blk, 32), lambda i: (i, 0))
                   for _ in range(4)] +
                  [pl.BlockSpec((blk, 16), lambda i: (i, 0))],
        out_shape=[jax.ShapeDtypeStruct((e, 32), F32) for _ in range(4)] +
                  [jax.ShapeDtypeStruct((e, 16), F32)],
    )(rka, rq, rva, smat, tmat)


def _finalize(x, parts, wout, bout, skip, trailing_gelu, blk=400):
    n = x.shape[0]
    ne = len(parts)
    t16 = jnp.asarray(_T16)

    def body(*refs):
        x_ref = refs[0]
        t_ref = refs[1]
        w_ref = refs[2]
        b_ref = refs[3]
        sk_ref = refs[4]
        agg = None
        for e in range(ne):
            h0, h1, h2, h3, d0, d1 = (refs[5 + 6 * e + j][0]
                                      for j in range(6))
            denb = jnp.dot(d0 + d1, t_ref[...], precision=PREC) + 1e-16
            numer = jnp.concatenate([h0, h1, h2, h3], axis=1)
            a = numer / denb
            agg = a if agg is None else agg + a
        h = jax.nn.gelu(agg)
        out = jnp.dot(h, w_ref[...], precision=PREC) + b_ref[...]
        sk = jax.nn.sigmoid(sk_ref[0, 0])
        res = sk * out + (1.0 - sk) * x_ref[...]
        if trailing_gelu:
            res = jax.nn.gelu(res)
        refs[-1][...] = res

    in_specs = [
        pl.BlockSpec((blk, D), lambda i: (i, 0)),
        pl.BlockSpec((16, D), lambda i: (0, 0)),
        pl.BlockSpec((D, D), lambda i: (0, 0)),
        pl.BlockSpec((1, D), lambda i: (0, 0)),
        pl.BlockSpec((1, 1), lambda i: (0, 0)),
    ]
    args = [x, t16, wout, bout, skip]
    for numer_a, numer_b, den in parts:
        in_specs += [
            pl.BlockSpec((1, blk, 32), lambda i: (0, i, 0)),
            pl.BlockSpec((1, blk, 32), lambda i: (0, i, 0)),
            pl.BlockSpec((1, blk, 32), lambda i: (1, i, 0)),
            pl.BlockSpec((1, blk, 32), lambda i: (1, i, 0)),
            pl.BlockSpec((1, blk, 16), lambda i: (0, i, 0)),
            pl.BlockSpec((1, blk, 16), lambda i: (1, i, 0)),
        ]
        args += [numer_a, numer_b, numer_a, numer_b, den, den]

    return pl.pallas_call(
        body,
        grid=(n // blk,),
        in_specs=in_specs,
        out_specs=pl.BlockSpec((blk, D), lambda i: (i, 0)),
        out_shape=jax.ShapeDtypeStruct((n, D), F32),
    )(*args)


def _pool(x, batch, nb=64, blk=400):
    n = x.shape[0]
    batch2 = batch.reshape(n, 1)

    def body(x_ref, b_ref, o_ref):
        i = pl.program_id(0)

        @pl.when(i == 0)
        def _():
            o_ref[...] = jnp.full((nb, D), -jnp.inf, F32)

        ids = b_ref[...]
        xv = x_ref[...]
        for b in range(nb):
            sel = jnp.where(ids == b, xv, -jnp.inf)
            red = jnp.max(sel, axis=0, keepdims=True)
            o_ref[b:b + 1, :] = jnp.maximum(o_ref[b:b + 1, :], red)

    return pl.pallas_call(
        body,
        grid=(n // blk,),
        in_specs=[
            pl.BlockSpec((blk, D), lambda i: (i, 0)),
            pl.BlockSpec((blk, 1), lambda i: (i, 0)),
        ],
        out_specs=pl.BlockSpec((nb, D), lambda i: (0, 0)),
        out_shape=jax.ShapeDtypeStruct((nb, D), F32),
    )(x, batch2)


@functools.cache
def _mesh():
    return plsc.VectorSubcoreMesh(core_axis_name="c", subcore_axis_name="s",
                                  num_cores=NC, num_subcores=NS)


def _gather3(ka_t, q_t, va_t, src, dst):
    e = src.shape[0]
    nchunks = e // CH
    iters = nchunks // NW
    assert nchunks % NW == 0

    @functools.partial(
        pl.kernel,
        mesh=_mesh(),
        out_type=[jax.ShapeDtypeStruct((e, D), F32) for _ in range(3)],
        scratch_types=[
            pltpu.VMEM((CH,), jnp.int32),
            pltpu.VMEM((CH,), jnp.int32),
            pltpu.VMEM((CH, D), F32),
            pltpu.VMEM((CH, D), F32),
            pltpu.VMEM((CH, D), F32),
            pltpu.SemaphoreType.DMA,
            pltpu.SemaphoreType.DMA,
            pltpu.SemaphoreType.DMA,
        ],
    )
    def k(ka_hbm, q_hbm, va_hbm, src_hbm, dst_hbm, oka, oq, ova,
          idx_s, idx_d, r1, r2, r3, s1, s2, s3):
        wid = lax.axis_index("s") * NC + lax.axis_index("c")

        @pl.loop(0, iters)
        def _(i):
            base = (wid + i * NW) * CH
            pltpu.sync_copy(src_hbm.at[pl.ds(base, CH)], idx_s)
            pltpu.sync_copy(dst_hbm.at[pl.ds(base, CH)], idx_d)
            c1 = pltpu.async_copy(ka_hbm.at[idx_s], r1, s1)
            c2 = pltpu.async_copy(q_hbm.at[idx_d], r2, s2)
            c3 = pltpu.async_copy(va_hbm.at[idx_s], r3, s3)
            c1.wait()
            c2.wait()
            c3.wait()
            pltpu.sync_copy(r1, oka.at[pl.ds(base, CH)])
            pltpu.sync_copy(r2, oq.at[pl.ds(base, CH)])
            pltpu.sync_copy(r3, ova.at[pl.ds(base, CH)])

    return k(ka_t, q_t, va_t, src, dst)


def _scatter_feat(contrib_a, contrib_b, dst, n_dst):
    e = dst.shape[0]
    nchunks = e // CH
    iters = nchunks // NS
    assert nchunks % NS == 0
    nz = n_dst // NS

    @functools.partial(
        pl.kernel,
        mesh=_mesh(),
        out_type=jax.ShapeDtypeStruct((NC * n_dst, 32), F32),
        scratch_types=[
            pltpu.VMEM_SHARED((n_dst, 32), F32),
            pltpu.VMEM((ZC, 32), F32),
            pltpu.VMEM((CH, 32), F32),
            pltpu.VMEM((CH,), jnp.int32),
        ],
    )
    def k(ca_hbm, cb_hbm, dst_hbm, out_hbm, accum, buf, rows_v, idx_v):
        core = lax.axis_index("c")
        sub = lax.axis_index("s")
        off = core * n_dst + sub * nz

        @pl.loop(0, ZC)
        def _(r):
            for c in range(2):
                buf[r, pl.ds(c * 16, 16)] = jnp.zeros((16,), F32)

        @pl.loop(0, nz // ZC)
        def _(j):
            pltpu.sync_copy(buf, accum.at[pl.ds(sub * nz + j * ZC, ZC)])

        plsc.subcore_barrier()

        def process(src_hbm):
            @pl.loop(0, iters)
            def _(i):
                base = (sub + i * NS) * CH
                pltpu.sync_copy(dst_hbm.at[pl.ds(base, CH)], idx_v)
                pltpu.sync_copy(src_hbm.at[pl.ds(base, CH)], rows_v)
                pltpu.sync_copy(rows_v, accum.at[idx_v], add=True)

        @pl.when(core == 0)
        def _():
            process(ca_hbm)

        @pl.when(core == 1)
        def _():
            process(cb_hbm)

        plsc.subcore_barrier()

        @pl.loop(0, nz // ZC)
        def _(j):
            pltpu.sync_copy(accum.at[pl.ds(sub * nz + j * ZC, ZC)], buf)
            pltpu.sync_copy(buf, out_hbm.at[pl.ds(off + j * ZC, ZC)])

    return k(contrib_a, contrib_b, dst).reshape(NC, n_dst, 32)


def _scatter_den(ex, dst, n_dst):
    e = dst.shape[0]
    nchunks = e // CH
    iters = nchunks // NW
    assert nchunks % NW == 0
    nz = n_dst // NS

    @functools.partial(
        pl.kernel,
        mesh=_mesh(),
        out_type=jax.ShapeDtypeStruct((NC * n_dst, 16), F32),
        scratch_types=[
            pltpu.VMEM_SHARED((n_dst, 16), F32),
            pltpu.VMEM((ZC, 16), F32),
            pltpu.VMEM((CH, 16), F32),
            pltpu.VMEM((CH,), jnp.int32),
        ],
    )
    def k(ex_hbm, dst_hbm, out_hbm, accum, buf, rows_v, idx_v):
        core = lax.axis_index("c")
        sub = lax.axis_index("s")
        wid = sub * NC + core
        off = core * n_dst + sub * nz

        @pl.loop(0, ZC)
        def _(r):
            buf[r, pl.ds(0, 16)] = jnp.zeros((16,), F32)

        @pl.loop(0, nz // ZC)
        def _(j):
            pltpu.sync_copy(buf, accum.at[pl.ds(sub * nz + j * ZC, ZC)])

        plsc.subcore_barrier()

        @pl.loop(0, iters)
        def _(i):
            base = (wid + i * NW) * CH
            pltpu.sync_copy(dst_hbm.at[pl.ds(base, CH)], idx_v)
            pltpu.sync_copy(ex_hbm.at[pl.ds(base, CH)], rows_v)
            pltpu.sync_copy(rows_v, accum.at[idx_v], add=True)

        plsc.subcore_barrier()

        @pl.loop(0, nz // ZC)
        def _(j):
            pltpu.sync_copy(accum.at[pl.ds(sub * nz + j * ZC, ZC)], buf)
            pltpu.sync_copy(buf, out_hbm.at[pl.ds(off + j * ZC, ZC)])

    return k(ex, dst).reshape(NC, n_dst, 16)


EP = 163840


def _edge_pass(ka_t, q_t, va_t, src, dst, n_dst, n_real, pvec):
    pad = EP - src.shape[0]
    src = jnp.concatenate([src, jnp.zeros((pad,), jnp.int32)])
    dst = jnp.concatenate([dst, jnp.full((pad,), n_real, jnp.int32)])
    rka, rq, rva = _gather3(ka_t, q_t, va_t, src, dst)
    scal = jnp.concatenate([pvec / np.sqrt(DH), jnp.zeros((12,), F32)])
    smat = jnp.asarray(_MASK16) * scal[None, :]
    c0, c1, c2, c3, ex = _edge_compute(rka, rq, rva, smat, jnp.asarray(_T16))
    za = jnp.zeros((n_dst, 32), F32)
    numer_a = jnp.stack([za.at[dst].add(c0), za.at[dst].add(c2)])
    numer_b = jnp.stack([za.at[dst].add(c1), za.at[dst].add(c3)])
    zd = jnp.zeros((n_dst, 16), F32)
    den = jnp.stack([zd.at[dst].add(ex), zd])
    return numer_a, numer_b, den


def kernel(piece_x, square_x, params, ei_ps, ei_sp, ei_ss, piece_batch,
           square_batch):
    c1, c2 = params["conv1"], params["conv2"]

    sq_pad = jnp.pad(square_x, ((0, 0), (0, 4)))
    wsq = jnp.pad(params["lin_square"]["w"], ((0, 4), (0, 0)))
    (x1p,) = _proj(piece_x, params["lin_piece"]["w"],
                   params["lin_piece"]["b"].reshape(1, D), 1, True)
    (x1s,) = _proj(sq_pad, wsq, params["lin_square"]["b"].reshape(1, D),
                   1, True)

    trip = [
        (c1["k"]["piece"], c1["rel"]["ps"]["a"]),
        (c1["v"]["piece"], c1["rel"]["ps"]["m"]),
        (c1["k"]["square"], c1["rel"]["sp"]["a"]),
        (c1["v"]["square"], c1["rel"]["sp"]["m"]),
        (c1["k"]["square"], c1["rel"]["ss"]["a"]),
        (c1["v"]["square"], c1["rel"]["ss"]["m"]),
        (c2["k"]["square"], c2["rel"]["sp"]["a"]),
        (c2["v"]["square"], c2["rel"]["sp"]["m"]),
    ]
    wstack = jnp.stack([lin["w"] for lin, _ in trip])
    bstack = jnp.stack([lin["b"].reshape(1, D) for lin, _ in trip])
    astack = jnp.stack([a for _, a in trip])
    wc, bc = _combine(wstack, bstack, astack)

    wcat_p = jnp.concatenate([c1["q"]["piece"]["w"], wc[0], wc[1]], axis=1)
    bcat_p = jnp.concatenate(
        [c1["q"]["piece"]["b"].reshape(1, D), bc[0], bc[1]], axis=1)
    q1p, ka_ps, va_ps = _proj(x1p, wcat_p, bcat_p, 3, False)
    wcat_s = jnp.concatenate(
        [c1["q"]["square"]["w"], wc[2], wc[3], wc[4], wc[5]], axis=1)
    bcat_s = jnp.concatenate(
        [c1["q"]["square"]["b"].reshape(1, D), bc[2], bc[3], bc[4], bc[5]],
        axis=1)
    q1s, ka_sp, va_sp, ka_ss, va_ss = _proj(x1s, wcat_s, bcat_s, 5, False)

    parts_ps = _edge_pass(ka_ps, q1s, va_ps, ei_ps[0], ei_ps[1], 32000,
                          30000, c1["rel"]["ps"]["p"])
    parts_sp = _edge_pass(ka_sp, q1p, va_sp, ei_sp[0], ei_sp[1], 22400,
                          20000, c1["rel"]["sp"]["p"])
    parts_ss = _edge_pass(ka_ss, q1s, va_ss, ei_ss[0], ei_ss[1], 32000,
                          30000, c1["rel"]["ss"]["p"])

    z_p = _finalize(x1p, [parts_sp], c1["out"]["piece"]["w"],
                    c1["out"]["piece"]["b"].reshape(1, D),
                    c1["skip"]["piece"].reshape(1, 1), True)
    z_s = _finalize(x1s, [parts_ps, parts_ss], c1["out"]["square"]["w"],
                    c1["out"]["square"]["b"].reshape(1, D),
                    c1["skip"]["square"].reshape(1, 1), True)

    (q2p,) = _proj(z_p, c2["q"]["piece"]["w"],
                   c2["q"]["piece"]["b"].reshape(1, D), 1, False)
    wcat2 = jnp.concatenate([wc[6], wc[7]], axis=1)
    bcat2 = jnp.concatenate([bc[6], bc[7]], axis=1)
    ka2, va2 = _proj(z_s, wcat2, bcat2, 2, False)
    parts2 = _edge_pass(ka2, q2p, va2, ei_sp[0], ei_sp[1], 22400,
                        20000, c2["rel"]["sp"]["p"])
    x2p = _finalize(z_p, [parts2], c2["out"]["piece"]["w"],
                    c2["out"]["piece"]["b"].reshape(1, D),
                    c2["skip"]["piece"].reshape(1, 1), False)

    return _pool(x2p, piece_batch)


# speedup vs baseline: 7.9761x; 1.0002x over previous
import functools

import numpy as np
import jax
import jax.numpy as jnp
from jax import lax
from jax.experimental import pallas as pl
from jax.experimental.pallas import tpu as pltpu
from jax.experimental.pallas import tpu_sc as plsc

F32 = jnp.float32
H = 4
DH = 32
D = 128
NC, NS = 2, 16
NW = NC * NS
CH = 128
PREC = lax.Precision.HIGHEST

_MASK16 = np.zeros((D, 16), np.float32)
for _h in range(H):
    _MASK16[_h * DH:(_h + 1) * DH, _h] = 1.0
_T16 = np.zeros((16, D), np.float32)
for _h in range(H):
    _T16[_h, _h * DH:(_h + 1) * DH] = 1.0


def _proj(x, w, b, nout, gelu, blk=400):
    n, kdim = x.shape

    def body(x_ref, w_ref, b_ref, *out_refs):
        acc = jnp.dot(x_ref[...], w_ref[...], precision=PREC) + b_ref[...]
        if gelu:
            acc = jax.nn.gelu(acc)
        for j, o in enumerate(out_refs):
            o[...] = acc[:, j * D:(j + 1) * D]

    return pl.pallas_call(
        body,
        grid=(n // blk,),
        in_specs=[
            pl.BlockSpec((blk, kdim), lambda i: (i, 0)),
            pl.BlockSpec((kdim, nout * D), lambda i: (0, 0)),
            pl.BlockSpec((1, nout * D), lambda i: (0, 0)),
        ],
        out_specs=[pl.BlockSpec((blk, D), lambda i: (i, 0))
                   for _ in range(nout)],
        out_shape=[jax.ShapeDtypeStruct((n, D), F32) for _ in range(nout)],
    )(x, w, b)


def _combine(wstack, bstack, astack):
    s = wstack.shape[0]

    def body(w_ref, b_ref, a_ref, ow_ref, ob_ref):
        for h in range(H):
            sl = slice(h * DH, (h + 1) * DH)
            a = a_ref[0, h]
            ow_ref[0, :, sl] = jnp.dot(w_ref[0][:, sl], a, precision=PREC)
            ob_ref[0, :, sl] = jnp.dot(b_ref[0][:, sl], a, precision=PREC)

    return pl.pallas_call(
        body,
        grid=(s,),
        in_specs=[
            pl.BlockSpec((1, D, D), lambda i: (i, 0, 0)),
            pl.BlockSpec((1, 1, D), lambda i: (i, 0, 0)),
            pl.BlockSpec((1, H, DH, DH), lambda i: (i, 0, 0, 0)),
        ],
        out_specs=[
            pl.BlockSpec((1, D, D), lambda i: (i, 0, 0)),
            pl.BlockSpec((1, 1, D), lambda i: (i, 0, 0)),
        ],
        out_shape=[jax.ShapeDtypeStruct((s, D, D), F32),
                   jax.ShapeDtypeStruct((s, 1, D), F32)],
    )(wstack, bstack, astack)


def _edge_compute(rka, rq, rva, smat, tmat, blk=640):
    e = rka.shape[0]

    def body(ka_ref, q_ref, va_ref, s_ref, t_ref, c0_ref, c1_ref, c2_ref,
             c3_ref, ex_ref):
        prod = ka_ref[...] * q_ref[...]
        alpha = jnp.dot(prod, s_ref[...], precision=PREC)
        ex = jnp.exp(alpha)
        exb = jnp.dot(ex, t_ref[...], precision=PREC)
        contrib = va_ref[...] * exb
        c0_ref[...] = contrib[:, 0:32]
        c1_ref[...] = contrib[:, 32:64]
        c2_ref[...] = contrib[:, 64:96]
        c3_ref[...] = contrib[:, 96:128]
        ex_ref[...] = ex

    return pl.pallas_call(
        body,
        grid=(e // blk,),
        in_specs=[
            pl.BlockSpec((blk, D), lambda i: (i, 0)),
            pl.BlockSpec((blk, D), lambda i: (i, 0)),
            pl.BlockSpec((blk, D), lambda i: (i, 0)),
            pl.BlockSpec((D, 16), lambda i: (0, 0)),
            pl.BlockSpec((16, D), lambda i: (0, 0)),
        ],
        out_specs=[pl.BlockSpec((blk, 32), lambda i: (i, 0))
                   for _ in range(4)] +
                  [pl.BlockSpec((blk, 16), lambda i: (i, 0))],
        out_shape=[jax.ShapeDtypeStruct((e, 32), F32) for _ in range(4)] +
                  [jax.ShapeDtypeStruct((e, 16), F32)],
    )(rka, rq, rva, smat, tmat)


def _finalize(x, parts, wout, bout, skip, trailing_gelu, blk=400):
    n = x.shape[0]
    ne = len(parts)
    t16 = jnp.asarray(_T16)

    def body(*refs):
        x_ref = refs[0]
        t_ref = refs[1]
        w_ref = refs[2]
        b_ref = refs[3]
        sk_ref = refs[4]
        agg = None
        for e in range(ne):
            h0, h1, h2, h3, d0, d1 = (refs[5 + 6 * e + j][0]
                                      for j in range(6))
            denb = jnp.dot(d0 + d1, t_ref[...], precision=PREC) + 1e-16
            numer = jnp.concatenate([h0, h1, h2, h3], axis=1)
            a = numer / denb
            agg = a if agg is None else agg + a
        h = jax.nn.gelu(agg)
        out = jnp.dot(h, w_ref[...], precision=PREC) + b_ref[...]
        sk = jax.nn.sigmoid(sk_ref[0, 0])
        res = sk * out + (1.0 - sk) * x_ref[...]
        if trailing_gelu:
            res = jax.nn.gelu(res)
        refs[-1][...] = res

    in_specs = [
        pl.BlockSpec((blk, D), lambda i: (i, 0)),
        pl.BlockSpec((16, D), lambda i: (0, 0)),
        pl.BlockSpec((D, D), lambda i: (0, 0)),
        pl.BlockSpec((1, D), lambda i: (0, 0)),
        pl.BlockSpec((1, 1), lambda i: (0, 0)),
    ]
    args = [x, t16, wout, bout, skip]
    for numer_a, numer_b, den in parts:
        in_specs += [
            pl.BlockSpec((1, blk, 32), lambda i: (0, i, 0)),
            pl.BlockSpec((1, blk, 32), lambda i: (0, i, 0)),
            pl.BlockSpec((1, blk, 32), lambda i: (1, i, 0)),
            pl.BlockSpec((1, blk, 32), lambda i: (1, i, 0)),
            pl.BlockSpec((1, blk, 16), lambda i: (0, i, 0)),
            pl.BlockSpec((1, blk, 16), lambda i: (1, i, 0)),
        ]
        args += [numer_a, numer_b, numer_a, numer_b, den, den]

    return pl.pallas_call(
        body,
        grid=(n // blk,),
        in_specs=in_specs,
        out_specs=pl.BlockSpec((blk, D), lambda i: (i, 0)),
        out_shape=jax.ShapeDtypeStruct((n, D), F32),
    )(*args)


def _pool(x, batch, nb=64, blk=400):
    n = x.shape[0]
    batch2 = batch.reshape(n, 1)

    def body(x_ref, b_ref, o_ref):
        i = pl.program_id(0)

        @pl.when(i == 0)
        def _():
            o_ref[...] = jnp.full((nb, D), -jnp.inf, F32)

        ids = b_ref[...]
        xv = x_ref[...]
        for b in range(nb):
            sel = jnp.where(ids == b, xv, -jnp.inf)
            red = jnp.max(sel, axis=0, keepdims=True)
            o_ref[b:b + 1, :] = jnp.maximum(o_ref[b:b + 1, :], red)

    return pl.pallas_call(
        body,
        grid=(n // blk,),
        in_specs=[
            pl.BlockSpec((blk, D), lambda i: (i, 0)),
            pl.BlockSpec((blk, 1), lambda i: (i, 0)),
        ],
        out_specs=pl.BlockSpec((nb, D), lambda i: (0, 0)),
        out_shape=jax.ShapeDtypeStruct((nb, D), F32),
    )(x, batch2)


@functools.cache
def _mesh():
    return plsc.VectorSubcoreMesh(core_axis_name="c", subcore_axis_name="s",
                                  num_cores=NC, num_subcores=NS)


def _gather3(ka_t, q_t, va_t, src, dst):
    e = src.shape[0]
    nchunks = e // CH
    iters = nchunks // NW
    assert nchunks % NW == 0

    @functools.partial(
        pl.kernel,
        mesh=_mesh(),
        out_type=[jax.ShapeDtypeStruct((e, D), F32) for _ in range(3)],
        scratch_types=[
            pltpu.VMEM((CH,), jnp.int32),
            pltpu.VMEM((CH,), jnp.int32),
            pltpu.VMEM((CH, D), F32),
            pltpu.VMEM((CH, D), F32),
            pltpu.VMEM((CH, D), F32),
            pltpu.SemaphoreType.DMA,
            pltpu.SemaphoreType.DMA,
            pltpu.SemaphoreType.DMA,
        ],
    )
    def k(ka_hbm, q_hbm, va_hbm, src_hbm, dst_hbm, oka, oq, ova,
          idx_s, idx_d, r1, r2, r3, s1, s2, s3):
        wid = lax.axis_index("s") * NC + lax.axis_index("c")

        @pl.loop(0, iters)
        def _(i):
            base = (wid + i * NW) * CH
            pltpu.sync_copy(src_hbm.at[pl.ds(base, CH)], idx_s)
            pltpu.sync_copy(dst_hbm.at[pl.ds(base, CH)], idx_d)
            c1 = pltpu.async_copy(ka_hbm.at[idx_s], r1, s1)
            c2 = pltpu.async_copy(q_hbm.at[idx_d], r2, s2)
            c3 = pltpu.async_copy(va_hbm.at[idx_s], r3, s3)
            c1.wait()
            c2.wait()
            c3.wait()
            pltpu.sync_copy(r1, oka.at[pl.ds(base, CH)])
            pltpu.sync_copy(r2, oq.at[pl.ds(base, CH)])
            pltpu.sync_copy(r3, ova.at[pl.ds(base, CH)])

    return k(ka_t, q_t, va_t, src, dst)


EP = 163840


def _edge_pass(ka_t, q_t, va_t, src, dst, n_dst, n_real, pvec):
    pad = EP - src.shape[0]
    src = jnp.concatenate([src, jnp.zeros((pad,), jnp.int32)])
    dst = jnp.concatenate([dst, jnp.full((pad,), n_real, jnp.int32)])
    rka, rq, rva = _gather3(ka_t, q_t, va_t, src, dst)
    scal = jnp.concatenate([pvec / np.sqrt(DH), jnp.zeros((12,), F32)])
    smat = jnp.asarray(_MASK16) * scal[None, :]
    c0, c1, c2, c3, ex = _edge_compute(rka, rq, rva, smat, jnp.asarray(_T16))
    za = jnp.zeros((n_dst, 32), F32)
    numer_a = jnp.stack([za.at[dst].add(c0), za.at[dst].add(c2)])
    numer_b = jnp.stack([za.at[dst].add(c1), za.at[dst].add(c3)])
    zd = jnp.zeros((n_dst, 16), F32)
    den = jnp.stack([zd.at[dst].add(ex), zd])
    return numer_a, numer_b, den


def kernel(piece_x, square_x, params, ei_ps, ei_sp, ei_ss, piece_batch,
           square_batch):
    c1, c2 = params["conv1"], params["conv2"]

    sq_pad = jnp.pad(square_x, ((0, 0), (0, 4)))
    wsq = jnp.pad(params["lin_square"]["w"], ((0, 4), (0, 0)))
    (x1p,) = _proj(piece_x, params["lin_piece"]["w"],
                   params["lin_piece"]["b"].reshape(1, D), 1, True)
    (x1s,) = _proj(sq_pad, wsq, params["lin_square"]["b"].reshape(1, D),
                   1, True)

    trip = [
        (c1["k"]["piece"], c1["rel"]["ps"]["a"]),
        (c1["v"]["piece"], c1["rel"]["ps"]["m"]),
        (c1["k"]["square"], c1["rel"]["sp"]["a"]),
        (c1["v"]["square"], c1["rel"]["sp"]["m"]),
        (c1["k"]["square"], c1["rel"]["ss"]["a"]),
        (c1["v"]["square"], c1["rel"]["ss"]["m"]),
        (c2["k"]["square"], c2["rel"]["sp"]["a"]),
        (c2["v"]["square"], c2["rel"]["sp"]["m"]),
    ]
    wstack = jnp.stack([lin["w"] for lin, _ in trip])
    bstack = jnp.stack([lin["b"].reshape(1, D) for lin, _ in trip])
    astack = jnp.stack([a for _, a in trip])
    wc, bc = _combine(wstack, bstack, astack)

    wcat_p = jnp.concatenate([c1["q"]["piece"]["w"], wc[0], wc[1]], axis=1)
    bcat_p = jnp.concatenate(
        [c1["q"]["piece"]["b"].reshape(1, D), bc[0], bc[1]], axis=1)
    q1p, ka_ps, va_ps = _proj(x1p, wcat_p, bcat_p, 3, False)
    wcat_s = jnp.concatenate(
        [c1["q"]["square"]["w"], wc[2], wc[3], wc[4], wc[5]], axis=1)
    bcat_s = jnp.concatenate(
        [c1["q"]["square"]["b"].reshape(1, D), bc[2], bc[3], bc[4], bc[5]],
        axis=1)
    q1s, ka_sp, va_sp, ka_ss, va_ss = _proj(x1s, wcat_s, bcat_s, 5, False)

    parts_ps = _edge_pass(ka_ps, q1s, va_ps, ei_ps[0], ei_ps[1], 32000,
                          30000, c1["rel"]["ps"]["p"])
    parts_sp = _edge_pass(ka_sp, q1p, va_sp, ei_sp[0], ei_sp[1], 22400,
                          20000, c1["rel"]["sp"]["p"])
    parts_ss = _edge_pass(ka_ss, q1s, va_ss, ei_ss[0], ei_ss[1], 32000,
                          30000, c1["rel"]["ss"]["p"])

    z_p = _finalize(x1p, [parts_sp], c1["out"]["piece"]["w"],
                    c1["out"]["piece"]["b"].reshape(1, D),
                    c1["skip"]["piece"].reshape(1, 1), True)
    z_s = _finalize(x1s, [parts_ps, parts_ss], c1["out"]["square"]["w"],
                    c1["out"]["square"]["b"].reshape(1, D),
                    c1["skip"]["square"].reshape(1, 1), True)

    (q2p,) = _proj(z_p, c2["q"]["piece"]["w"],
                   c2["q"]["piece"]["b"].reshape(1, D), 1, False)
    wcat2 = jnp.concatenate([wc[6], wc[7]], axis=1)
    bcat2 = jnp.concatenate([bc[6], bc[7]], axis=1)
    ka2, va2 = _proj(z_s, wcat2, bcat2, 2, False)
    parts2 = _edge_pass(ka2, q2p, va2, ei_sp[0], ei_sp[1], 22400,
                        20000, c2["rel"]["sp"]["p"])
    x2p = _finalize(z_p, [parts2], c2["out"]["piece"]["w"],
                    c2["out"]["piece"]["b"].reshape(1, D),
                    c2["skip"]["piece"].reshape(1, 1), False)

    return _pool(x2p, piece_batch)
